# baseline (speedup 1.0000x reference)
import functools

import jax
import jax.numpy as jnp
import numpy as np
from jax import lax
from jax.experimental import pallas as pl
from jax.experimental.pallas import tpu as pltpu
from jax.experimental.pallas import tpu_sc as plsc

_TABLE_SIZES = [100000] * 26
_NF = len(_TABLE_SIZES)
_OUT_DIM = 64
_BATCH = 16384
_B_FLAT = _BATCH * _NF
_NW = 32
_PER_W = _B_FLAT // _NW
_CHUNK = 128
_NCHUNK = _PER_W // _CHUNK
_NBUF = 4

assert _B_FLAT % _NW == 0 and _PER_W % _CHUNK == 0 and _PER_W % _NF == 0

_OFFSETS = np.cumsum([0] + _TABLE_SIZES[:-1]).astype(np.int32)
_OFF_TILE = np.tile(_OFFSETS, _PER_W // _NF).reshape(_NCHUNK, _CHUNK)

_mesh = plsc.VectorSubcoreMesh(core_axis_name="c", subcore_axis_name="s")


@functools.partial(
    pl.kernel,
    mesh=_mesh,
    out_type=jax.ShapeDtypeStruct((_B_FLAT, _OUT_DIM), jnp.float32),
    scratch_types=[
        pltpu.VMEM((_NCHUNK, _CHUNK), jnp.int32),
        pltpu.VMEM((_NCHUNK, _CHUNK), jnp.int32),
        pltpu.VMEM((_NBUF, _CHUNK, _OUT_DIM), jnp.float32),
    ]
    + [pltpu.SemaphoreType.DMA] * (2 * _NBUF),
    compiler_params=pltpu.CompilerParams(use_tc_tiling_on_sc=False),
)
def _sc_gather(idx_hbm, off_hbm, table_hbm, out_hbm, idx_v, off_v, rows_v,
               *sems):
    gsems, ssems = sems[:_NBUF], sems[_NBUF:]
    wid = lax.axis_index("s") * 2 + lax.axis_index("c")
    base = wid * _PER_W

    pltpu.sync_copy(idx_hbm.at[wid], idx_v)
    pltpu.sync_copy(off_hbm, off_v)

    def _shift(c, carry):
        for j in range(_CHUNK // 16):
            s = pl.ds(j * 16, 16)
            idx_v[c, s] = idx_v[c, s] + off_v[c, s]
        return carry

    lax.fori_loop(0, _NCHUNK, _shift, 0)

    def _group(g, carry):
        c0 = g * _NBUF
        gathers = []
        for b in range(_NBUF):
            gathers.append(pltpu.async_copy(
                table_hbm.at[idx_v.at[c0 + b]], rows_v.at[b], gsems[b]))
        stores = []
        for b in range(_NBUF):
            gathers[b].wait()
            stores.append(pltpu.async_copy(
                rows_v.at[b],
                out_hbm.at[pl.ds(base + (c0 + b) * _CHUNK, _CHUNK)],
                ssems[b]))
        for b in range(_NBUF):
            stores[b].wait()
        return carry

    lax.fori_loop(0, _NCHUNK // _NBUF, _group, 0)


def kernel(indices, table):
    idx3 = indices.reshape(_NW, _NCHUNK, _CHUNK)
    off = jnp.asarray(_OFF_TILE)
    out = _sc_gather(idx3, off, table)
    return out.reshape(_BATCH, _NF, _OUT_DIM)


# speedup vs baseline: 1.0045x; 1.0045x over previous
import functools

import jax
import jax.numpy as jnp
import numpy as np
from jax import lax
from jax.experimental import pallas as pl
from jax.experimental.pallas import tpu as pltpu
from jax.experimental.pallas import tpu_sc as plsc

_TABLE_SIZES = [100000] * 26
_NF = len(_TABLE_SIZES)
_OUT_DIM = 64
_BATCH = 16384
_B_FLAT = _BATCH * _NF
_NW = 32
_PER_W = _B_FLAT // _NW
_CHUNK = 128
_NCHUNK = _PER_W // _CHUNK
_NBUF = 8
_LAG = 4

assert _B_FLAT % _NW == 0 and _PER_W % _CHUNK == 0 and _PER_W % _NF == 0

_OFFSETS = np.cumsum([0] + _TABLE_SIZES[:-1]).astype(np.int32)
_OFF_TILE = np.tile(_OFFSETS, _PER_W // _NF).reshape(_NCHUNK, _CHUNK)

_mesh = plsc.VectorSubcoreMesh(core_axis_name="c", subcore_axis_name="s")


@functools.partial(
    pl.kernel,
    mesh=_mesh,
    out_type=jax.ShapeDtypeStruct((_B_FLAT, _OUT_DIM), jnp.float32),
    scratch_types=[
        pltpu.VMEM((_NCHUNK, _CHUNK), jnp.int32),
        pltpu.VMEM((_NCHUNK, _CHUNK), jnp.int32),
        pltpu.VMEM((_NBUF, _CHUNK, _OUT_DIM), jnp.float32),
    ]
    + [pltpu.SemaphoreType.DMA] * (2 * _NBUF),
    compiler_params=pltpu.CompilerParams(use_tc_tiling_on_sc=False),
)
def _sc_gather(idx_hbm, off_hbm, table_hbm, out_hbm, idx_v, off_v, rows_v,
               *sems):
    gsems, ssems = sems[:_NBUF], sems[_NBUF:]
    wid = lax.axis_index("s") * 2 + lax.axis_index("c")
    base = wid * _PER_W

    pltpu.sync_copy(idx_hbm.at[wid], idx_v)
    pltpu.sync_copy(off_hbm, off_v)

    def _shift(c, carry):
        for j in range(_CHUNK // 16):
            s = pl.ds(j * 16, 16)
            idx_v[c, s] = idx_v[c, s] + off_v[c, s]
        return carry

    lax.fori_loop(0, _NCHUNK, _shift, 0)

    def _g_start(c, b):
        pltpu.async_copy(table_hbm.at[idx_v.at[c]], rows_v.at[b], gsems[b])

    def _g_wait(c, b):
        pltpu.make_async_copy(table_hbm.at[idx_v.at[c]], rows_v.at[b],
                              gsems[b]).wait()

    def _s_start(c, b):
        pltpu.async_copy(
            rows_v.at[b], out_hbm.at[pl.ds(base + c * _CHUNK, _CHUNK)],
            ssems[b])

    def _s_wait(c, b):
        pltpu.make_async_copy(
            rows_v.at[b], out_hbm.at[pl.ds(base + c * _CHUNK, _CHUNK)],
            ssems[b]).wait()

    for b in range(_NBUF):
        _g_start(b, b)
    for b in range(_NBUF - _LAG):
        _g_wait(b, b)
        _s_start(b, b)

    def _group(g, carry):
        c0 = g * _NBUF
        for b in range(_NBUF):
            c = c0 + b
            _s_wait(c - _NBUF, b)
            _g_start(c, b)
            bl = (b - _LAG) % _NBUF
            _g_wait(c - _LAG, bl)
            _s_start(c - _LAG, bl)
        return carry

    lax.fori_loop(1, _NCHUNK // _NBUF, _group, 0)

    for i in range(_LAG):
        c = _NCHUNK - _LAG + i
        b = c % _NBUF
        _g_wait(c, b)
        _s_start(c, b)
    for i in range(_NBUF):
        c = _NCHUNK - _NBUF + i
        b = c % _NBUF
        _s_wait(c, b)


def kernel(indices, table):
    idx3 = indices.reshape(_NW, _NCHUNK, _CHUNK)
    off = jnp.asarray(_OFF_TILE)
    out = _sc_gather(idx3, off, table)
    return out.reshape(_BATCH, _NF, _OUT_DIM)
